# baseline (speedup 1.0000x reference)
import jax
import jax.numpy as jnp
from jax import lax
from jax.experimental import pallas as pl
from jax.experimental.pallas import tpu as pltpu
from jax.experimental.pallas import tpu_sc as plsc

N = 10000
E = 160000
D = 256
HALF = 128
NC = 2
NS = 16
NPAD = 10240
SEG = NPAD // NS
EPT_DEG = E // (NC * NS)
EPT = E // NS
CH = 80
NCH = EPT // CH
APAD = 10112
RPT = APAD // NS
BN = 400

_MESH = plsc.VectorSubcoreMesh(core_axis_name="c", subcore_axis_name="s")
_SC_PARAMS = pltpu.CompilerParams(needs_layout_passes=False)


def _deg_body(dst_hbm, out_hbm, deg_sh, hist, dbuf, sbuf, acc):
    c = lax.axis_index("c")
    s = lax.axis_index("s")
    tid = c * NS + s

    def zero(i, carry):
        hist[pl.ds(i * 16, 16)] = jnp.zeros((16,), jnp.float32)
        return carry

    lax.fori_loop(0, NPAD // 16, zero, 0)

    pltpu.sync_copy(dst_hbm.at[pl.ds(tid * EPT_DEG, EPT_DEG)],
                    dbuf.at[pl.ds(0, EPT_DEG)])
    lane = lax.iota(jnp.int32, 16)
    tail = dbuf[pl.ds(EPT_DEG - 8, 16)]
    dbuf[pl.ds(EPT_DEG - 8, 16)] = jnp.where(lane < 8, tail, N)

    ones = jnp.ones((16,), jnp.float32)

    def count(i, carry):
        idx = dbuf[pl.ds(i * 16, 16)]
        plsc.addupdate_scatter(hist, [idx], ones)
        return carry

    lax.fori_loop(0, (EPT_DEG + 8) // 16, count, 0)

    pltpu.sync_copy(hist, deg_sh.at[s])
    plsc.subcore_barrier()
    for t in range(NS):
        pltpu.sync_copy(deg_sh.at[t, pl.ds(s * SEG, SEG)], sbuf.at[t])

    def reduce(j, carry):
        a = sbuf[0, pl.ds(j * 16, 16)]
        for t in range(1, NS):
            a = a + sbuf[t, pl.ds(j * 16, 16)]
        acc[pl.ds(j * 16, 16)] = a
        return carry

    lax.fori_loop(0, SEG // 16, reduce, 0)
    pltpu.sync_copy(acc, out_hbm.at[c, pl.ds(s * SEG, SEG)])


_deg_call = pl.kernel(
    _deg_body,
    out_type=jax.ShapeDtypeStruct((NC, NPAD), jnp.float32),
    mesh=_MESH,
    compiler_params=_SC_PARAMS,
    scratch_types=[
        pltpu.VMEM_SHARED((NS, NPAD), jnp.float32),
        pltpu.VMEM((NPAD,), jnp.float32),
        pltpu.VMEM((EPT_DEG + 16,), jnp.int32),
        pltpu.VMEM((NS, SEG), jnp.float32),
        pltpu.VMEM((SEG,), jnp.float32),
    ],
)


def _round_body(g_hbm, sd_hbm, zeros_hbm, out_hbm, agg_sh,
                sd0, sd1, sd2, sd3, sd4, sd5, rows0, rows1, rows2,
                sg0, sg1, sg2, ss0, ss1, ss2, si0, si1, si2, si3, si4, si5):
    c = lax.axis_index("c")
    s = lax.axis_index("s")

    pltpu.sync_copy(zeros_hbm, agg_sh.at[pl.ds(s * RPT, RPT)])

    sd = (sd0, sd1, sd2, sd3, sd4, sd5)
    rows = (rows0, rows1, rows2)
    sg = (sg0, sg1, sg2)
    ss = (ss0, ss1, ss2)
    si = (si0, si1, si2, si3, si4, si5)

    def i_issue(i, q):
        pltpu.async_copy(sd_hbm.at[c, s, pl.ds(i, 1)], sd[q], si[q])

    def i_wait(i, q):
        pltpu.make_async_copy(sd_hbm.at[c, s, pl.ds(i, 1)], sd[q], si[q]).wait()

    def g_issue(b, q):
        pltpu.async_copy(g_hbm.at[sd[q].at[0, 0]], rows[b], sg[b])

    def g_wait(b, q):
        pltpu.make_async_copy(g_hbm.at[sd[q].at[0, 0]], rows[b], sg[b]).wait()

    def scat_issue(b, q):
        pltpu.async_copy(rows[b], agg_sh.at[sd[q].at[0, 1]], ss[b], add=True)

    def scat_wait(b, q):
        pltpu.make_async_copy(rows[b], agg_sh.at[sd[q].at[0, 1]], ss[b]).wait()

    plsc.subcore_barrier()

    pltpu.sync_copy(sd_hbm.at[c, s, pl.ds(0, 1)], sd0)
    g_issue(0, 0)
    i_issue(1, 1)

    def step(i, b, q, s_wait_prev, issue_next, wait_next_idx=True):
        b1 = (b + 1) % 3
        q1 = (q + 1) % 6
        q2 = (q + 2) % 6
        b2 = (b + 1) % 3
        g_wait(b, q)
        if wait_next_idx:
            i_wait(i + 1, q1)
        if s_wait_prev:
            scat_wait((b + 1) % 3, (q + 4) % 6)
        if wait_next_idx:
            g_issue(b1, q1)
        scat_issue(b, q)
        if issue_next:
            i_issue(i + 2, q2)

    step(0, 0, 0, False, True)
    step(1, 1, 1, False, True)

    def hexa(k, carry):
        i = 6 * k + 2
        step(i, 2, 2, True, True)
        step(i + 1, 0, 3, True, True)
        step(i + 2, 1, 4, True, True)
        step(i + 3, 2, 5, True, True)
        step(i + 4, 0, 0, True, True)
        step(i + 5, 1, 1, True, True)
        return carry

    lax.fori_loop(0, (NCH - 5) // 6, hexa, 0)
    step(NCH - 3, 2, 2, True, True)
    step(NCH - 2, 0, 3, True, False)
    step(NCH - 1, 1, 4, True, False, wait_next_idx=False)
    scat_wait(0, 3)
    scat_wait(1, 4)
    plsc.subcore_barrier()
    pltpu.sync_copy(agg_sh.at[pl.ds(s * RPT, RPT)],
                    out_hbm.at[pl.ds(c * APAD + s * RPT, RPT)])


_round_call = pl.kernel(
    _round_body,
    out_type=jax.ShapeDtypeStruct((NC * APAD, HALF), jnp.float32),
    mesh=_MESH,
    compiler_params=_SC_PARAMS,
    scratch_types=(
        [pltpu.VMEM_SHARED((APAD, HALF), jnp.float32)]
        + [pltpu.VMEM((1, 2, CH), jnp.int32) for _ in range(6)]
        + [pltpu.VMEM((CH, HALF), jnp.float32) for _ in range(3)]
        + [pltpu.SemaphoreType.DMA for _ in range(12)]
    ),
)


def _tc_b_body(feat_ref, wt_ref, degp_ref, g1_ref, ri_ref, nrm_ref, niv_ref):
    x = feat_ref[...]
    h0 = lax.dot_general(x, wt_ref[...], (((1,), (0,)), ((), ())),
                         preferred_element_type=jnp.float32)
    dp = degp_ref[...]
    deg = jnp.maximum(dp[:, 0:1] + dp[:, 1:2], 1.0)
    norm = lax.rsqrt(deg)
    ninv = 1.0 / deg
    g1 = h0 * norm
    ri = h0 * ninv
    g1_ref[0] = g1[:, :HALF]
    g1_ref[1] = g1[:, HALF:]
    ri_ref[0] = ri[:, :HALF]
    ri_ref[1] = ri[:, HALF:]
    nrm_ref[...] = norm
    niv_ref[...] = ninv


_tc_b_call = pl.pallas_call(
    _tc_b_body,
    grid=(N // BN,),
    in_specs=[
        pl.BlockSpec((BN, D), lambda i: (i, 0)),
        pl.BlockSpec((D, D), lambda i: (0, 0)),
        pl.BlockSpec((BN, 2), lambda i: (i, 0)),
    ],
    out_specs=[
        pl.BlockSpec((NC, BN, HALF), lambda i: (0, i, 0)),
        pl.BlockSpec((NC, BN, HALF), lambda i: (0, i, 0)),
        pl.BlockSpec((BN, 1), lambda i: (i, 0)),
        pl.BlockSpec((BN, 1), lambda i: (i, 0)),
    ],
    out_shape=[
        jax.ShapeDtypeStruct((NC, N, HALF), jnp.float32),
        jax.ShapeDtypeStruct((NC, N, HALF), jnp.float32),
        jax.ShapeDtypeStruct((N, 1), jnp.float32),
        jax.ShapeDtypeStruct((N, 1), jnp.float32),
    ],
)


def _tc_d_body(agg_ref, g1_ref, niv_ref, g2_ref):
    nv = niv_ref[...][None]
    g2_ref[...] = (agg_ref[...] + g1_ref[...]) * nv


_tc_d_call = pl.pallas_call(
    _tc_d_body,
    grid=(N // BN,),
    in_specs=[
        pl.BlockSpec((NC, BN, HALF), lambda i: (0, i, 0)),
        pl.BlockSpec((NC, BN, HALF), lambda i: (0, i, 0)),
        pl.BlockSpec((BN, 1), lambda i: (i, 0)),
    ],
    out_specs=pl.BlockSpec((NC, BN, HALF), lambda i: (0, i, 0)),
    out_shape=jax.ShapeDtypeStruct((NC, N, HALF), jnp.float32),
)


def _tc_f_body(agg_ref, ri_ref, nrm_ref, out_ref):
    nm = nrm_ref[...]
    a = agg_ref[...]
    r = ri_ref[...]
    out_ref[:, :HALF] = a[0] * nm + r[0]
    out_ref[:, HALF:] = a[1] * nm + r[1]


_tc_f_call = pl.pallas_call(
    _tc_f_body,
    grid=(N // BN,),
    in_specs=[
        pl.BlockSpec((NC, BN, HALF), lambda i: (0, i, 0)),
        pl.BlockSpec((NC, BN, HALF), lambda i: (0, i, 0)),
        pl.BlockSpec((BN, 1), lambda i: (i, 0)),
    ],
    out_specs=pl.BlockSpec((BN, D), lambda i: (i, 0)),
    out_shape=jax.ShapeDtypeStruct((N, D), jnp.float32),
)


def kernel(features, edge_index, W):
    src = edge_index[0]
    dst = edge_index[1]
    wt = W.T
    dstr = dst.reshape(NS, NCH, CH)
    sd = jnp.stack([
        jnp.stack([(src + cc * N).reshape(NS, NCH, CH), dstr], axis=2)
        for cc in range(NC)], axis=0)
    zeros = jnp.zeros((RPT, HALF), jnp.float32)
    degp = _deg_call(dst)
    degp_t = degp.T
    g1, ri, nrm, niv = _tc_b_call(features, wt, degp_t)
    agg1 = _round_call(g1.reshape(NC * N, HALF), sd, zeros)
    g2 = _tc_d_call(agg1.reshape(NC, APAD, HALF), g1, niv)
    agg2 = _round_call(g2.reshape(NC * N, HALF), sd, zeros)
    out = _tc_f_call(agg2.reshape(NC, APAD, HALF), ri, nrm)
    return out


# speedup vs baseline: 1.0018x; 1.0018x over previous
import jax
import jax.numpy as jnp
from jax import lax
from jax.experimental import pallas as pl
from jax.experimental.pallas import tpu as pltpu
from jax.experimental.pallas import tpu_sc as plsc

N = 10000
E = 160000
D = 256
HALF = 128
NC = 2
NS = 16
NPAD = 10240
SEG = NPAD // NS
EPT_DEG = E // (NC * NS)
EPT = E // NS
CH = 80
NCH = EPT // CH
APAD = 10112
RPT = APAD // NS
BN = 400

_MESH = plsc.VectorSubcoreMesh(core_axis_name="c", subcore_axis_name="s")
_SC_PARAMS = pltpu.CompilerParams(needs_layout_passes=False)


def _deg_body(dst_hbm, out_hbm, deg_sh, hist, dbuf, sbuf, acc):
    c = lax.axis_index("c")
    s = lax.axis_index("s")
    tid = c * NS + s

    def zero(i, carry):
        hist[pl.ds(i * 16, 16)] = jnp.zeros((16,), jnp.float32)
        return carry

    lax.fori_loop(0, NPAD // 16, zero, 0)

    pltpu.sync_copy(dst_hbm.at[pl.ds(tid * EPT_DEG, EPT_DEG)],
                    dbuf.at[pl.ds(0, EPT_DEG)])
    lane = lax.iota(jnp.int32, 16)
    tail = dbuf[pl.ds(EPT_DEG - 8, 16)]
    dbuf[pl.ds(EPT_DEG - 8, 16)] = jnp.where(lane < 8, tail, N)

    ones = jnp.ones((16,), jnp.float32)

    def count(i, carry):
        idx = dbuf[pl.ds(i * 16, 16)]
        plsc.addupdate_scatter(hist, [idx], ones)
        return carry

    lax.fori_loop(0, (EPT_DEG + 8) // 16, count, 0)

    pltpu.sync_copy(hist, deg_sh.at[s])
    plsc.subcore_barrier()
    for t in range(NS):
        pltpu.sync_copy(deg_sh.at[t, pl.ds(s * SEG, SEG)], sbuf.at[t])

    def reduce(j, carry):
        a = sbuf[0, pl.ds(j * 16, 16)]
        for t in range(1, NS):
            a = a + sbuf[t, pl.ds(j * 16, 16)]
        acc[pl.ds(j * 16, 16)] = a
        return carry

    lax.fori_loop(0, SEG // 16, reduce, 0)
    pltpu.sync_copy(acc, out_hbm.at[c, pl.ds(s * SEG, SEG)])


_deg_call = pl.kernel(
    _deg_body,
    out_type=jax.ShapeDtypeStruct((NC, NPAD), jnp.float32),
    mesh=_MESH,
    compiler_params=_SC_PARAMS,
    scratch_types=[
        pltpu.VMEM_SHARED((NS, NPAD), jnp.float32),
        pltpu.VMEM((NPAD,), jnp.float32),
        pltpu.VMEM((EPT_DEG + 16,), jnp.int32),
        pltpu.VMEM((NS, SEG), jnp.float32),
        pltpu.VMEM((SEG,), jnp.float32),
    ],
)


def _round_body(g_hbm, sd_hbm, zeros_hbm, out_hbm, agg_sh,
                sd0, sd1, sd2, sd3, rows0, rows1,
                sg0, sg1, ss0, ss1, si0, si1, si2, si3):
    c = lax.axis_index("c")
    s = lax.axis_index("s")

    pltpu.sync_copy(zeros_hbm, agg_sh.at[pl.ds(s * RPT, RPT)])

    sd = (sd0, sd1, sd2, sd3)
    rows = (rows0, rows1)
    sg = (sg0, sg1)
    ss = (ss0, ss1)
    si = (si0, si1, si2, si3)

    def i_issue(i, q):
        pltpu.async_copy(sd_hbm.at[c, s, pl.ds(i, 1)], sd[q], si[q])

    def i_wait(i, q):
        pltpu.make_async_copy(sd_hbm.at[c, s, pl.ds(i, 1)], sd[q], si[q]).wait()

    def g_issue(b, q):
        pltpu.async_copy(g_hbm.at[sd[q].at[0, 0]], rows[b], sg[b])

    def g_wait(b, q):
        pltpu.make_async_copy(g_hbm.at[sd[q].at[0, 0]], rows[b], sg[b]).wait()

    def scat_issue(b, q):
        pltpu.async_copy(rows[b], agg_sh.at[sd[q].at[0, 1]], ss[b], add=True)

    def scat_wait(b, q):
        pltpu.make_async_copy(rows[b], agg_sh.at[sd[q].at[0, 1]], ss[b]).wait()

    plsc.subcore_barrier()

    pltpu.sync_copy(sd_hbm.at[c, s, pl.ds(0, 1)], sd0)
    g_issue(0, 0)
    i_issue(1, 1)

    def step(i, b, q, s_wait_prev, issue_next):
        nb = 1 - b
        q1 = (q + 1) % 4
        q2 = (q + 2) % 4
        q3 = (q + 3) % 4
        g_wait(b, q)
        i_wait(i + 1, q1)
        if s_wait_prev:
            scat_wait(nb, q3)
        g_issue(nb, q1)
        scat_issue(b, q)
        if issue_next:
            i_issue(i + 2, q2)

    step(0, 0, 0, False, True)
    step(1, 1, 1, True, True)

    def quad(k, carry):
        i = 4 * k + 2
        step(i, 0, 2, True, True)
        step(i + 1, 1, 3, True, True)
        step(i + 2, 0, 0, True, True)
        step(i + 3, 1, 1, True, True)
        return carry

    lax.fori_loop(0, (NCH - 5) // 4, quad, 0)
    step(NCH - 3, 0, 2, True, True)
    step(NCH - 2, 1, 3, True, False)
    g_wait(0, 0)
    scat_wait(1, 3)
    scat_issue(0, 0)
    scat_wait(0, 0)
    plsc.subcore_barrier()
    pltpu.sync_copy(agg_sh.at[pl.ds(s * RPT, RPT)],
                    out_hbm.at[pl.ds(c * APAD + s * RPT, RPT)])


_round_call = pl.kernel(
    _round_body,
    out_type=jax.ShapeDtypeStruct((NC * APAD, HALF), jnp.float32),
    mesh=_MESH,
    compiler_params=_SC_PARAMS,
    scratch_types=(
        [pltpu.VMEM_SHARED((APAD, HALF), jnp.float32)]
        + [pltpu.VMEM((1, 2, CH), jnp.int32) for _ in range(4)]
        + [pltpu.VMEM((CH, HALF), jnp.float32) for _ in range(2)]
        + [pltpu.SemaphoreType.DMA for _ in range(8)]
    ),
)


def _tc_b_body(feat_ref, wt_ref, degp_ref, g1_ref, ri_ref, nrm_ref, niv_ref):
    x = feat_ref[...]
    h0 = lax.dot_general(x, wt_ref[...], (((1,), (0,)), ((), ())),
                         preferred_element_type=jnp.float32)
    dp = degp_ref[...]
    deg = jnp.maximum(dp[:, 0:1] + dp[:, 1:2], 1.0)
    norm = lax.rsqrt(deg)
    ninv = 1.0 / deg
    g1 = h0 * norm
    ri = h0 * ninv
    g1_ref[0] = g1[:, :HALF]
    g1_ref[1] = g1[:, HALF:]
    ri_ref[0] = ri[:, :HALF]
    ri_ref[1] = ri[:, HALF:]
    nrm_ref[...] = norm
    niv_ref[...] = ninv


_tc_b_call = pl.pallas_call(
    _tc_b_body,
    grid=(N // BN,),
    in_specs=[
        pl.BlockSpec((BN, D), lambda i: (i, 0)),
        pl.BlockSpec((D, D), lambda i: (0, 0)),
        pl.BlockSpec((BN, 2), lambda i: (i, 0)),
    ],
    out_specs=[
        pl.BlockSpec((NC, BN, HALF), lambda i: (0, i, 0)),
        pl.BlockSpec((NC, BN, HALF), lambda i: (0, i, 0)),
        pl.BlockSpec((BN, 1), lambda i: (i, 0)),
        pl.BlockSpec((BN, 1), lambda i: (i, 0)),
    ],
    out_shape=[
        jax.ShapeDtypeStruct((NC, N, HALF), jnp.float32),
        jax.ShapeDtypeStruct((NC, N, HALF), jnp.float32),
        jax.ShapeDtypeStruct((N, 1), jnp.float32),
        jax.ShapeDtypeStruct((N, 1), jnp.float32),
    ],
)


def _tc_d_body(agg_ref, g1_ref, niv_ref, g2_ref):
    nv = niv_ref[...][None]
    g2_ref[...] = (agg_ref[...] + g1_ref[...]) * nv


_tc_d_call = pl.pallas_call(
    _tc_d_body,
    grid=(N // BN,),
    in_specs=[
        pl.BlockSpec((NC, BN, HALF), lambda i: (0, i, 0)),
        pl.BlockSpec((NC, BN, HALF), lambda i: (0, i, 0)),
        pl.BlockSpec((BN, 1), lambda i: (i, 0)),
    ],
    out_specs=pl.BlockSpec((NC, BN, HALF), lambda i: (0, i, 0)),
    out_shape=jax.ShapeDtypeStruct((NC, N, HALF), jnp.float32),
)


def _tc_f_body(agg_ref, ri_ref, nrm_ref, out_ref):
    nm = nrm_ref[...]
    a = agg_ref[...]
    r = ri_ref[...]
    out_ref[:, :HALF] = a[0] * nm + r[0]
    out_ref[:, HALF:] = a[1] * nm + r[1]


_tc_f_call = pl.pallas_call(
    _tc_f_body,
    grid=(N // BN,),
    in_specs=[
        pl.BlockSpec((NC, BN, HALF), lambda i: (0, i, 0)),
        pl.BlockSpec((NC, BN, HALF), lambda i: (0, i, 0)),
        pl.BlockSpec((BN, 1), lambda i: (i, 0)),
    ],
    out_specs=pl.BlockSpec((BN, D), lambda i: (i, 0)),
    out_shape=jax.ShapeDtypeStruct((N, D), jnp.float32),
)


def kernel(features, edge_index, W):
    src = edge_index[0]
    dst = edge_index[1]
    wt = W.T
    dstr = dst.reshape(NS, NCH, CH)
    sd = jnp.stack([
        jnp.stack([(src + cc * N).reshape(NS, NCH, CH), dstr], axis=2)
        for cc in range(NC)], axis=0)
    zeros = jnp.zeros((RPT, HALF), jnp.float32)
    degp = _deg_call(dst)
    degp_t = degp.T
    g1, ri, nrm, niv = _tc_b_call(features, wt, degp_t)
    agg1 = _round_call(g1.reshape(NC * N, HALF), sd, zeros)
    g2 = _tc_d_call(agg1.reshape(NC, APAD, HALF), g1, niv)
    agg2 = _round_call(g2.reshape(NC * N, HALF), sd, zeros)
    out = _tc_f_call(agg2.reshape(NC, APAD, HALF), ri, nrm)
    return out
